# Initial kernel scaffold; baseline (speedup 1.0000x reference)
import jax
import jax.numpy as jnp
from jax.experimental import pallas as pl


def kernel(x, weight):
    raise NotImplementedError("write your pallas kernel here")


# speedup vs baseline: 2.4963x; 2.4963x over previous
import jax
import jax.numpy as jnp
from jax import lax
from jax.experimental import pallas as pl
from jax.experimental.pallas import tpu as pltpu
from jax.experimental.pallas import tpu_sc as plsc

_NUM_EMB = 100000
_D = 128
_EMBED_SCALE = 11.313708498984761
_QMAX = 127.0
_MAGIC = 12582912.0

_NC, _NS = 2, 16
_NW = _NC * _NS
_B = 4096 * 50
_PER_W = _B // _NW
_CHUNK = 128
_NCHUNK = _PER_W // _CHUNK

_AMAX_BLOCK = 2000


def _amax_body(w_ref, o_ref):
    i = pl.program_id(0)
    m = jnp.max(jnp.abs(w_ref[...]))

    @pl.when(i == 0)
    def _():
        o_ref[0, 0] = m

    @pl.when(i > 0)
    def _():
        o_ref[0, 0] = jnp.maximum(o_ref[0, 0], m)


def _table_amax(weight):
    return pl.pallas_call(
        _amax_body,
        grid=(_NUM_EMB // _AMAX_BLOCK,),
        in_specs=[pl.BlockSpec((_AMAX_BLOCK, _D), lambda i: (i, 0))],
        out_specs=pl.BlockSpec(memory_space=pltpu.SMEM),
        out_shape=jax.ShapeDtypeStruct((1, 1), jnp.float32),
    )(weight)


def _sc_body(table_h, xf_h, par_h, out_h,
             idx_v, buf0, buf1, par_v, gs0, gs1, os0, os1):
    wid = lax.axis_index("s") * _NC + lax.axis_index("c")
    pltpu.sync_copy(xf_h.at[wid], idx_v)
    pltpu.sync_copy(par_h, par_v)
    inv = par_v[0, :]
    comb = par_v[1, :]
    base = wid * _PER_W

    bufs = (buf0, buf1)
    gsems = (gs0, gs1)
    osems = (os0, os1)
    gather_h = [None, None]
    store_h = [None, None]

    def compute(buf):
        def row_body(r, carry):
            for j in range(_D // 16):
                v = buf[r, pl.ds(j * 16, 16)]
                t = v * inv
                t = (t + _MAGIC) - _MAGIC
                t = jnp.minimum(jnp.maximum(t, -_QMAX - 1.0), _QMAX)
                buf[r, pl.ds(j * 16, 16)] = t * comb
            return carry

        lax.fori_loop(0, _CHUNK, row_body, 0)

    gather_h[0] = pltpu.async_copy(table_h.at[idx_v.at[0]], buf0, gs0)
    for c in range(_NCHUNK):
        s = c & 1
        ns = s ^ 1
        if c + 1 < _NCHUNK:
            if store_h[ns] is not None:
                store_h[ns].wait()
                store_h[ns] = None
            gather_h[ns] = pltpu.async_copy(
                table_h.at[idx_v.at[c + 1]], bufs[ns], gsems[ns])
        gather_h[s].wait()
        compute(bufs[s])
        store_h[s] = pltpu.async_copy(
            bufs[s], out_h.at[pl.ds(base + c * _CHUNK, _CHUNK)], osems[s])
    store_h[0].wait()
    store_h[1].wait()


def kernel(x, weight):
    weight = weight.astype(jnp.float32)
    amax = _table_amax(weight)[0, 0]
    scale = jnp.maximum(amax, 1e-8) / _QMAX
    inv = 1.0 / scale
    comb = scale * jnp.float32(_EMBED_SCALE)
    params = jnp.stack([jnp.full((16,), inv, jnp.float32),
                        jnp.full((16,), comb, jnp.float32)])
    xf = x.astype(jnp.int32).reshape(_NW, _NCHUNK, _CHUNK)

    sc = pl.kernel(
        _sc_body,
        out_type=jax.ShapeDtypeStruct((_B, _D), jnp.float32),
        mesh=plsc.VectorSubcoreMesh(core_axis_name="c", subcore_axis_name="s",
                                    num_cores=_NC, num_subcores=_NS),
        scratch_types=[
            pltpu.VMEM((_NCHUNK, _CHUNK), jnp.int32),
            pltpu.VMEM((_CHUNK, _D), jnp.float32),
            pltpu.VMEM((_CHUNK, _D), jnp.float32),
            pltpu.VMEM((2, 16), jnp.float32),
            pltpu.SemaphoreType.DMA,
            pltpu.SemaphoreType.DMA,
            pltpu.SemaphoreType.DMA,
            pltpu.SemaphoreType.DMA,
        ],
    )
    out = sc(weight, xf, params)
    return out.reshape(4096, 50, _D)
